# baseline (speedup 1.0000x reference)
import functools

import jax
import jax.numpy as jnp
from jax.experimental import pallas as pl
from jax.experimental.pallas import tpu as pltpu

BM = 1024
BN = 2048
LANES = 128


def _nn_kernel(a_ref, b_ref, dist_ref, idx_ref, mrun_ref, krun_ref, b2_ref,
               *, n_blocks):
    i = pl.program_id(0)
    j = pl.program_id(1)
    nch = BN // LANES

    a = a_ref[...]
    x = jax.lax.dot_general(
        a * -2.0, b_ref[...], (((1,), (1,)), ((), ())),
        preferred_element_type=jnp.float32,
    )

    @pl.when(i == 0)
    def _compute_b2():
        b = b_ref[...]
        b2 = jnp.sum((b * b).reshape(nch, LANES, b.shape[1]), axis=2)
        b2_ref[pl.ds(j * nch, nch), :] = b2

    b2blk = b2_ref[pl.ds(j * nch, nch), :]

    m = jnp.where(j == 0, jnp.float32(jnp.inf), mrun_ref[...])
    kk = jnp.where(j == 0, 0, krun_ref[...])
    for t in range(nch):
        c = x[:, t * LANES:(t + 1) * LANES] + b2blk[t:t + 1, :]
        better = c < m
        kk = jnp.where(better, j * nch + t, kk)
        m = jnp.minimum(c, m)

    @pl.when(j < n_blocks - 1)
    def _save():
        mrun_ref[...] = m
        krun_ref[...] = kk

    @pl.when(j == n_blocks - 1)
    def _finish():
        lane_arg = jnp.argmin(m, axis=1).astype(jnp.int32)
        row_min = jnp.min(m, axis=1)
        onehot = (jax.lax.broadcasted_iota(jnp.int32, (BM, LANES), 1)
                  == lane_arg[:, None])
        chunk = jnp.max(jnp.where(onehot, kk, 0), axis=1)
        a2 = jnp.sum(a * a, axis=1)
        dist_ref[...] = jnp.sqrt(jnp.maximum(row_min + a2, 0.0))[:, None]
        idx_ref[...] = (chunk * LANES + lane_arg)[:, None]


def kernel(desc1, desc2):
    m, k = desc1.shape
    n, _ = desc2.shape
    m_blocks = m // BM
    n_blocks = n // BN

    dists, idxs = pl.pallas_call(
        functools.partial(_nn_kernel, n_blocks=n_blocks),
        grid=(m_blocks, n_blocks),
        in_specs=[
            pl.BlockSpec((BM, k), lambda i, j: (i, 0)),
            pl.BlockSpec((BN, k), lambda i, j: (j, 0)),
        ],
        out_specs=[
            pl.BlockSpec((BM, 1), lambda i, j: (i, 0)),
            pl.BlockSpec((BM, 1), lambda i, j: (i, 0)),
        ],
        out_shape=[
            jax.ShapeDtypeStruct((m, 1), jnp.float32),
            jax.ShapeDtypeStruct((m, 1), jnp.int32),
        ],
        scratch_shapes=[
            pltpu.VMEM((BM, LANES), jnp.float32),
            pltpu.VMEM((BM, LANES), jnp.int32),
            pltpu.VMEM((n // LANES, LANES), jnp.float32),
        ],
    )(desc1, desc2)

    idxs_in_1 = jnp.arange(m, dtype=jnp.int32).reshape(-1, 1)
    matches_idxs = jnp.concatenate([idxs_in_1, idxs], axis=1)
    return (dists, matches_idxs)


# speedup vs baseline: 2.8282x; 2.8282x over previous
import jax
import jax.numpy as jnp
from jax.experimental import pallas as pl

BM = 1024
BN = 2048
LANES = 128


def _b2_kernel(b_ref, out_ref):
    b = b_ref[...]
    nch = out_ref.shape[0]
    out_ref[...] = jnp.sum((b * b).reshape(nch, LANES, b.shape[1]), axis=2)


def _nn_kernel(a_ref, b_ref, b2_ref, dist_ref, idx_ref):
    a = a_ref[...]
    b2 = b2_ref[...]
    n = b_ref.shape[0]
    nch = BN // LANES

    m = jnp.full((BM, LANES), jnp.inf, jnp.float32)
    kk = jnp.zeros((BM, LANES), jnp.int32)
    for j in range(n // BN):
        x = jax.lax.dot_general(
            a * -2.0, b_ref[j * BN:(j + 1) * BN, :],
            (((1,), (1,)), ((), ())),
            preferred_element_type=jnp.float32,
        )
        for t in range(nch):
            g = j * nch + t
            c = x[:, t * LANES:(t + 1) * LANES] + b2[g:g + 1, :]
            better = c < m
            kk = jnp.where(better, g, kk)
            m = jnp.minimum(c, m)

    lane_arg = jnp.argmin(m, axis=1).astype(jnp.int32)
    row_min = jnp.min(m, axis=1)
    onehot = (jax.lax.broadcasted_iota(jnp.int32, (BM, LANES), 1)
              == lane_arg[:, None])
    chunk = jnp.max(jnp.where(onehot, kk, 0), axis=1)
    a2 = jnp.sum(a * a, axis=1)
    dist_ref[...] = jnp.sqrt(jnp.maximum(row_min + a2, 0.0))[:, None]
    idx_ref[...] = (chunk * LANES + lane_arg)[:, None]


def kernel(desc1, desc2):
    m, k = desc1.shape
    n, _ = desc2.shape
    m_blocks = m // BM

    b2 = pl.pallas_call(
        _b2_kernel,
        out_shape=jax.ShapeDtypeStruct((n // LANES, LANES), jnp.float32),
    )(desc2)

    dists, idxs = pl.pallas_call(
        _nn_kernel,
        grid=(m_blocks,),
        in_specs=[
            pl.BlockSpec((BM, k), lambda i: (i, 0)),
            pl.BlockSpec((n, k), lambda i: (0, 0)),
            pl.BlockSpec((n // LANES, LANES), lambda i: (0, 0)),
        ],
        out_specs=[
            pl.BlockSpec((BM, 1), lambda i: (i, 0)),
            pl.BlockSpec((BM, 1), lambda i: (i, 0)),
        ],
        out_shape=[
            jax.ShapeDtypeStruct((m, 1), jnp.float32),
            jax.ShapeDtypeStruct((m, 1), jnp.int32),
        ],
    )(desc1, desc2, b2)

    idxs_in_1 = jnp.arange(m, dtype=jnp.int32).reshape(-1, 1)
    matches_idxs = jnp.concatenate([idxs_in_1, idxs], axis=1)
    return (dists, matches_idxs)
